# baseline (speedup 1.0000x reference)
import functools

import jax
import jax.numpy as jnp
from jax import lax
from jax.experimental import pallas as pl
from jax.experimental.pallas import tpu as pltpu
from jax.experimental.pallas import tpu_sc as plsc

_L = 16
_NC = 2
_NS = 16
_NW = _NC * _NS
_CHUNK = 128


@functools.lru_cache(maxsize=None)
def _build(n_pad, ncols):
    per_w = n_pad // _NW
    n_chunks = per_w // _CHUNK
    mesh = plsc.VectorSubcoreMesh(core_axis_name="c", subcore_axis_name="s")

    @functools.partial(
        pl.kernel,
        mesh=mesh,
        out_type=jax.ShapeDtypeStruct((_NW, per_w), jnp.float32),
        scratch_types=[
            pltpu.VMEM((per_w,), jnp.int32),
            pltpu.VMEM((per_w,), jnp.int32),
            pltpu.VMEM((n_chunks, _CHUNK), jnp.int32),
            pltpu.VMEM((per_w,), jnp.float32),
            pltpu.SemaphoreType.DMA,
        ],
    )
    def gather_kernel(xflat, rows_hbm, cols_hbm, out_hbm,
                      rows_v, cols_v, lin_v, vals_v, sem):
        wid = lax.axis_index("s") * _NC + lax.axis_index("c")
        base = wid * per_w
        pltpu.sync_copy(rows_hbm.at[pl.ds(base, per_w)], rows_v)
        pltpu.sync_copy(cols_hbm.at[pl.ds(base, per_w)], cols_v)
        for j in range(n_chunks):
            for i in range(_CHUNK // _L):
                c = j * _CHUNK + i * _L
                r = rows_v[pl.ds(c, _L)]
                cc = cols_v[pl.ds(c, _L)]
                lin_v[j, pl.ds(i * _L, _L)] = r * ncols + cc
        copies = [
            pltpu.async_copy(xflat.at[lin_v.at[j]],
                             vals_v.at[pl.ds(j * _CHUNK, _CHUNK)], sem)
            for j in range(n_chunks)
        ]
        for cp in copies:
            cp.wait()
        pltpu.sync_copy(vals_v, out_hbm.at[wid])

    return gather_kernel


def kernel(X, indices):
    n = indices.shape[0]
    tile = _NW * _CHUNK
    n_pad = ((n + tile - 1) // tile) * tile
    idx = jnp.pad(indices, ((0, n_pad - n), (0, 0)))
    out = _build(n_pad, X.shape[1])(
        X.reshape(-1), idx[:, 0], idx[:, 1])
    return out.reshape(-1)[:n].reshape(-1, 2)
